# baseline (speedup 1.0000x reference)
import dataclasses

import jax
import jax.numpy as jnp
from jax.experimental import pallas as pl
from jax.experimental.pallas import tpu as pltpu
from jax.experimental.pallas import tpu_sc as plsc

_SC_PARAMS = pltpu.CompilerParams()
if "needs_layout_passes" in pltpu.CompilerParams.__dataclass_fields__:
    _SC_PARAMS = dataclasses.replace(_SC_PARAMS, needs_layout_passes=False)

_K = 512
_BETA = 1.0
_F = 32768
_B = 64
_NSL = _F // 16
_NMX = _NSL // 16


def _to_f32_vec(c):
    bits = c ^ ((c >> 31) & jnp.int32(0x7FFFFFFF))
    return plsc.bitcast(bits, jnp.float32)


def _splat_i32(x):
    return jnp.full((16,), x, jnp.int32)


def _count_ge(buf, nsl, cand_f):
    def body(i, acc):
        v = buf[pl.ds(i * 16, 16)]
        return acc + jnp.where(v >= cand_f, 1, 0).astype(jnp.int32)
    acc = jax.lax.fori_loop(0, nsl, body, jnp.zeros((16,), jnp.int32))
    return jnp.sum(acc)


def _greedy(buf, nsl, nbits, k):
    cnt_pos = _count_ge(buf, nsl, jnp.zeros((16,), jnp.float32))
    t = jnp.where(cnt_pos >= k, jnp.int32(0), jnp.int32(-2147483648))

    def body(i, t):
        b = jnp.int32(30) - i
        cand = t | (jnp.int32(1) << b)
        cnt = _count_ge(buf, nsl, _to_f32_vec(_splat_i32(cand)))
        return jnp.where(cnt >= k, cand, t)

    return jax.lax.fori_loop(0, nbits, body, t)


def _sc_thresh(boosted):
    mesh = plsc.VectorSubcoreMesh(core_axis_name="c", subcore_axis_name="s")

    @pl.kernel(
        out_type=jax.ShapeDtypeStruct((_B, 16), jnp.float32),
        mesh=mesh,
        scratch_types=[
            pltpu.VMEM((_F,), jnp.float32),
            pltpu.VMEM((_F + 16,), jnp.float32),
            pltpu.VMEM((_NSL,), jnp.float32),
            pltpu.VMEM((16,), jnp.float32),
            pltpu.SemaphoreType.DMA,
        ],
        compiler_params=_SC_PARAMS,
    )
    def sc_kernel(x_hbm, o_hbm, row, cand, mx, tbuf, sem):
        gid = jax.lax.axis_index("c") * 16 + jax.lax.axis_index("s")

        @pl.loop(0, 2)
        def _(which):
            r = gid * 2 + which
            pltpu.async_copy(x_hbm.at[r], row, sem).wait()

            @pl.loop(0, _NMX)
            def _(p):
                def mbody(q, acc):
                    v = row[pl.ds((p * 16 + q) * 16, 16)]
                    return jnp.maximum(acc, v)
                acc = jax.lax.fori_loop(
                    0, 16, mbody, jnp.full((16,), -jnp.inf, jnp.float32))
                mx[pl.ds(p * 16, 16)] = acc

            t_low = _greedy(mx, _NMX, 13, _K)
            tlow_f = _to_f32_vec(_splat_i32(t_low))

            def cbody(s, off):
                v = row[pl.ds(s * 16, 16)]
                m = v >= tlow_f
                pos = plsc.cumsum(m.astype(jnp.int32))
                idx = off + pos - 1
                plsc.store_scatter(cand, [idx], v, mask=m)
                return off + plsc.all_reduce_population_count(m)
            off = jax.lax.fori_loop(0, _NSL, cbody, jnp.zeros((16,), jnp.int32))
            mtot = jnp.max(off)

            pad_idx = mtot + jax.lax.iota(jnp.int32, 16)
            plsc.store_scatter(cand, [pad_idx],
                               jnp.full((16,), -jnp.inf, jnp.float32))

            ncsl = (mtot + 15) // 16
            t = _greedy(cand, ncsl, 31, _K)
            tbuf[...] = _to_f32_vec(_splat_i32(t))
            pltpu.async_copy(tbuf, o_hbm.at[r], sem).wait()

    return sc_kernel(boosted)


def _boost_kernel(x_ref, dc_ref, out_ref):
    target = jnp.float32(_K / _F)
    boost = jnp.exp(_BETA * (target - dc_ref[...]))
    out_ref[...] = x_ref[...] * boost


def _mask_kernel(x_ref, dc_ref, t_ref, out_ref):
    target = jnp.float32(_K / _F)
    boost = jnp.exp(_BETA * (target - dc_ref[...]))
    boosted = x_ref[...] * boost
    out_ref[...] = jnp.where(boosted >= t_ref[...], x_ref[...],
                             jnp.float32(0.0))


def kernel(inputs, duty_cycle):
    b, f = inputs.shape
    dc2 = duty_cycle.reshape(1, f)
    boosted = pl.pallas_call(
        _boost_kernel,
        out_shape=jax.ShapeDtypeStruct((b, f), jnp.float32),
    )(inputs, dc2)
    thresh = _sc_thresh(boosted)
    return pl.pallas_call(
        _mask_kernel,
        out_shape=jax.ShapeDtypeStruct((b, f), jnp.float32),
    )(inputs, dc2, thresh[:, :1])


# speedup vs baseline: 2.9588x; 2.9588x over previous
import jax
import jax.numpy as jnp
from jax.experimental import pallas as pl
from jax.experimental.pallas import tpu as pltpu

_K = 512
_BETA = 1.0


def _to_f32(c):
    bits = c ^ ((c >> 31) & jnp.int32(0x7FFFFFFF))
    return jax.lax.bitcast_convert_type(bits, jnp.float32)


def _kwinner_kernel(x_ref, dc_ref, out_ref):
    x = x_ref[...]
    dc = dc_ref[...]
    units = x.shape[-1]
    target = jnp.float32(_K / units)
    boost = jnp.exp(_BETA * (target - dc))
    boosted = x * boost

    k = jnp.float32(_K)

    def count_ge(cand_f):
        flags = jnp.where(boosted >= cand_f, jnp.float32(1.0), jnp.float32(0.0))
        return jnp.sum(flags, axis=1, keepdims=True)

    cnt_pos = count_ge(jnp.float32(0.0))
    t0 = jnp.where(cnt_pos >= k, jnp.int32(0), jnp.int32(-2147483648))

    def step(t, b):
        cand = t | (jnp.int32(1) << b)
        cnt = count_ge(_to_f32(cand))
        return jnp.where(cnt >= k, cand, t)

    def body(i, t):
        b = jnp.int32(30) - i * 2
        return step(step(t, b), b - 1)

    t = jax.lax.fori_loop(0, 15, body, t0)
    t = step(t, jnp.int32(0))
    t_f = _to_f32(t)

    out_ref[...] = jnp.where(boosted >= t_f, x, jnp.float32(0.0))


def kernel(inputs, duty_cycle):
    b, f = inputs.shape
    dc2 = duty_cycle.reshape(1, f)
    return pl.pallas_call(
        _kwinner_kernel,
        out_shape=jax.ShapeDtypeStruct((b, f), jnp.float32),
    )(inputs, dc2)


# speedup vs baseline: 3.5986x; 1.2163x over previous
import jax
import jax.numpy as jnp
from jax.experimental import pallas as pl
from jax.experimental.pallas import tpu as pltpu

_K = 512
_BETA = 1.0


def _kwinner_kernel(x_ref, dc_ref, out_ref):
    x = x_ref[...]
    dc = dc_ref[...]
    bsz, units = x.shape
    target = jnp.float32(_K / units)
    boost = jnp.exp(_BETA * (target - dc))
    boosted = x * boost

    bits = jax.lax.bitcast_convert_type(boosted, jnp.int32)
    w = bits ^ ((bits >> 31) & jnp.int32(0x7FFFFFFF))

    k = jnp.int32(_K)
    nchunk = 16
    csz = units // nchunk

    def count_ge_i32(cand):
        flags = jnp.where(w >= cand, jnp.int32(1), jnp.int32(0))
        return jnp.sum(flags, axis=1, keepdims=True)

    cnt_pos = count_ge_i32(jnp.zeros((bsz, 1), jnp.int32))
    t = jnp.where(cnt_pos >= k, jnp.int32(0), jnp.int32(-2147483648))

    def phase(t, s):
        b0 = t >> s
        hs = w >> s
        hc = jnp.minimum(jnp.maximum(hs, b0 - 1), b0 + 32767)
        d16 = (hc - b0).astype(jnp.int16)

        def step(i, t):
            b = s + 14 - i
            cand = t | (jnp.int32(1) << b)
            c16 = ((cand >> s) - b0).astype(jnp.int16)
            flags = jnp.where(d16 >= c16, jnp.int16(1), jnp.int16(0))
            s1 = flags[:, 0:csz]
            for j in range(1, nchunk):
                s1 = s1 + flags[:, j * csz:(j + 1) * csz]
            cnt = jnp.sum(s1.astype(jnp.int32), axis=1, keepdims=True)
            return jnp.where(cnt >= k, cand, t)

        return jax.lax.fori_loop(0, 15, step, t)

    t = phase(t, 16)
    t = phase(t, 1)
    cand = t | jnp.int32(1)
    cnt = count_ge_i32(cand)
    t = jnp.where(cnt >= k, cand, t)

    out_ref[...] = jnp.where(w >= t, x, jnp.float32(0.0))


def kernel(inputs, duty_cycle):
    b, f = inputs.shape
    dc2 = duty_cycle.reshape(1, f)
    return pl.pallas_call(
        _kwinner_kernel,
        out_shape=jax.ShapeDtypeStruct((b, f), jnp.float32),
    )(inputs, dc2)


# speedup vs baseline: 3.7419x; 1.0398x over previous
import jax
import jax.numpy as jnp
from jax.experimental import pallas as pl
from jax.experimental.pallas import tpu as pltpu

_K = 512
_BETA = 1.0


def _kwinner_kernel(x_ref, dc_ref, out_ref):
    x = x_ref[...]
    dc = dc_ref[...]
    bsz, units = x.shape
    target = jnp.float32(_K / units)
    boost = jnp.exp(_BETA * (target - dc))
    boosted = x * boost

    bits = jax.lax.bitcast_convert_type(boosted, jnp.int32)
    w = bits ^ ((bits >> 31) & jnp.int32(0x7FFFFFFF))

    k = jnp.int32(_K)
    nchunk = 16
    csz = units // nchunk

    def count16(d16, c16):
        flags = jnp.where(d16 >= c16, jnp.int16(1), jnp.int16(0))
        s1 = flags[:, 0:csz]
        for j in range(1, nchunk):
            s1 = s1 + flags[:, j * csz:(j + 1) * csz]
        return jnp.sum(s1.astype(jnp.int32), axis=1, keepdims=True)

    d16a = (w >> 16).astype(jnp.int16)

    cnt_pos = count16(d16a, jnp.zeros((bsz, 1), jnp.int16))
    t = jnp.where(cnt_pos >= k, jnp.int32(0), jnp.int32(-2147483648))

    def step_a(i, t):
        b = 30 - i
        cand = t | (jnp.int32(1) << b)
        c16 = (cand >> 16).astype(jnp.int16)
        cnt = count16(d16a, c16)
        return jnp.where(cnt >= k, cand, t)

    t = jax.lax.fori_loop(0, 15, step_a, t)

    b0 = t >> 1
    hs = w >> 1
    hc = jnp.minimum(jnp.maximum(hs, b0 - 1), b0 + 32767)
    d16b = (hc - b0).astype(jnp.int16)

    def step_b(i, t):
        b = 15 - i
        cand = t | (jnp.int32(1) << b)
        c16 = ((cand >> 1) - b0).astype(jnp.int16)
        cnt = count16(d16b, c16)
        return jnp.where(cnt >= k, cand, t)

    t = jax.lax.fori_loop(0, 15, step_b, t)

    cand = t | jnp.int32(1)
    flags = jnp.where(w >= cand, jnp.int32(1), jnp.int32(0))
    cnt = jnp.sum(flags, axis=1, keepdims=True)
    t = jnp.where(cnt >= k, cand, t)

    out_ref[...] = jnp.where(w >= t, x, jnp.float32(0.0))


def kernel(inputs, duty_cycle):
    b, f = inputs.shape
    dc2 = duty_cycle.reshape(1, f)
    return pl.pallas_call(
        _kwinner_kernel,
        out_shape=jax.ShapeDtypeStruct((b, f), jnp.float32),
    )(inputs, dc2)
